# baseline (speedup 1.0000x reference)
import jax
import jax.numpy as jnp
from jax import lax
from jax.experimental import pallas as pl
from jax.experimental.pallas import tpu as pltpu
from jax.experimental.pallas import tpu_sc as plsc

H = 512
BS = 32
NUM_LAGS = 9
N = BS * NUM_LAGS
E = 1024
NC = 2
NS = 16
EPW = E // (NC * NS)
RC = 16


def _sc_build_adjacency(src_hbm, dst_hbm, zer_hbm, out_hbm,
                        src_v, dst_v, rows_v, acc_sh):
    cid = lax.axis_index("c")
    sid = lax.axis_index("s")
    wid = sid * NC + cid
    base = wid * EPW
    pltpu.sync_copy(zer_hbm, rows_v)
    off = pl.multiple_of(sid * RC, RC)
    pltpu.sync_copy(zer_hbm.at[pl.ds(0, RC)], acc_sh.at[pl.ds(off, RC)])

    @pl.when(sid < 2)
    def _zero_tail():
        off2 = pl.multiple_of(NS * RC + sid * RC, RC)
        pltpu.sync_copy(zer_hbm.at[pl.ds(0, RC)], acc_sh.at[pl.ds(off2, RC)])
    pltpu.sync_copy(src_hbm.at[pl.ds(base, EPW)], src_v)
    pltpu.sync_copy(dst_hbm.at[pl.ds(base, EPW)], dst_v)
    ones = jnp.full((16,), 1.0, jnp.float32)
    for half in range(EPW // 16):
        ridx = lax.iota(jnp.int32, 16) + half * 16
        cidx = src_v[pl.ds(half * 16, 16)]
        plsc.store_scatter(rows_v, [ridx, cidx], ones)
    plsc.subcore_barrier()
    pltpu.sync_copy(rows_v, acc_sh.at[dst_v], add=True)
    plsc.subcore_barrier()
    off3 = pl.multiple_of(sid * RC, RC)
    pltpu.sync_copy(acc_sh.at[pl.ds(off3, RC)],
                    out_hbm.at[cid, pl.ds(off3, RC)])

    @pl.when(sid < 2)
    def _write_tail():
        off4 = pl.multiple_of(NS * RC + sid * RC, RC)
        pltpu.sync_copy(acc_sh.at[pl.ds(off4, RC)],
                        out_hbm.at[cid, pl.ds(off4, RC)])


_sc_adjacency = pl.kernel(
    _sc_build_adjacency,
    out_type=jax.ShapeDtypeStruct((NC, N, N), jnp.float32),
    mesh=plsc.VectorSubcoreMesh(core_axis_name="c", subcore_axis_name="s",
                                num_cores=NC, num_subcores=NS),
    compiler_params=pltpu.CompilerParams(use_tc_tiling_on_sc=False,
                                         needs_layout_passes=False),
    scratch_types=[
        pltpu.VMEM((EPW,), jnp.int32),
        pltpu.VMEM((EPW,), jnp.int32),
        pltpu.VMEM((EPW, N), jnp.float32),
        pltpu.VMEM_SHARED((N, N), jnp.float32),
    ],
)


def _fused_body(ap_ref, l_ref, w_ref, w1l_ref, b1l_ref, g_ref, bb_ref,
                w2l_ref, b2l_ref, w1w_ref, b1w_ref, w2w_ref, b2w_ref,
                m0w_ref, m0b_ref, m1w_ref, m1b_ref, regr_ref, regb_ref,
                out_ref):
    f32 = jnp.float32
    dot = lambda a, b: jax.lax.dot_general(
        a, b, (((1,), (0,)), ((), ())), preferred_element_type=f32,
        precision=jax.lax.Precision.HIGHEST)

    a = ap_ref[0] + ap_ref[1]
    deg = jnp.maximum(jnp.sum(a, axis=1, keepdims=True), 1.0)
    an = a / deg

    inv = 1.0 / jnp.sqrt(1.0 + 1e-5)
    l = jnp.tanh(l_ref[...] * w1l_ref[...] + b1l_ref[...])
    l = l * (g_ref[...] * inv) + bb_ref[...]
    l = jnp.tanh(dot(l, w2l_ref[...]) + b2l_ref[...])

    w = jnp.tanh(dot(w_ref[...], w1w_ref[...]) + b1w_ref[...])
    w = jnp.tanh(dot(w, w2w_ref[...]) + b2w_ref[...])

    ml = dot(an, l)
    mw = dot(an, w)
    l = dot(ml, m0w_ref[0]) + dot(mw, m0w_ref[1]) + m0b_ref[...]
    w = mw

    ml = dot(an, l)
    mw = dot(an, w)
    l = dot(ml, m1w_ref[0]) + dot(mw, m1w_ref[1]) + m1b_ref[...]

    s = jnp.sum(l * regr_ref[...], axis=1, keepdims=True)
    rows = jax.lax.broadcasted_iota(jnp.int32, (BS, N), 0)
    cols = jax.lax.broadcasted_iota(jnp.int32, (BS, N), 1)
    grp = (rows == cols // NUM_LAGS).astype(f32)
    out_ref[...] = dot(grp, s) + regb_ref[...]


def kernel(lags, weather, lags_W1, lags_b1, bn_g, bn_b, lags_W2, lags_b2,
           wea_W1, wea_b1, wea_W2, wea_b2, mgn0_W, mgn0_b, mgn1_W, mgn1_b,
           reg_W, reg_b, src, dst):
    f32 = jnp.float32
    zer = jnp.zeros((EPW, N), f32)
    ap = _sc_adjacency(src, dst, zer)
    l0 = lags.reshape(N, 1)
    w0 = weather.reshape(N, 8)
    reg_tiled = jnp.tile(reg_W.reshape(NUM_LAGS, H), (BS, 1))
    m0 = mgn0_W.reshape(2, H, H)
    m1 = mgn1_W.reshape(2, H, H)
    args = (
        ap, l0, w0, lags_W1, lags_b1.reshape(1, H), bn_g.reshape(1, H),
        bn_b.reshape(1, H), lags_W2, lags_b2.reshape(1, H), wea_W1,
        wea_b1.reshape(1, H), wea_W2, wea_b2.reshape(1, H), m0,
        mgn0_b.reshape(1, H), m1, mgn1_b.reshape(1, H), reg_tiled,
        reg_b.reshape(1, 1),
    )
    return pl.pallas_call(
        _fused_body,
        out_shape=jax.ShapeDtypeStruct((BS, 1), f32),
    )(*args)


# speedup vs baseline: 1.2446x; 1.2446x over previous
import jax
import jax.numpy as jnp
from jax import lax
from jax.experimental import pallas as pl
from jax.experimental.pallas import tpu as pltpu
from jax.experimental.pallas import tpu_sc as plsc

H = 512
BS = 32
NUM_LAGS = 9
N = BS * NUM_LAGS
E = 1024
NC = 2
NS = 16
EPW = E // (NC * NS)
RC = 16


def _sc_build_adjacency(src_hbm, dst_hbm, zer_hbm, out_hbm,
                        src_v, dst_v, rows_v, acc_sh):
    cid = lax.axis_index("c")
    sid = lax.axis_index("s")
    wid = sid * NC + cid
    base = wid * EPW
    pltpu.sync_copy(zer_hbm, rows_v)
    off = pl.multiple_of(sid * RC, RC)
    pltpu.sync_copy(zer_hbm.at[pl.ds(0, RC)], acc_sh.at[pl.ds(off, RC)])

    @pl.when(sid < 2)
    def _zero_tail():
        off2 = pl.multiple_of(NS * RC + sid * RC, RC)
        pltpu.sync_copy(zer_hbm.at[pl.ds(0, RC)], acc_sh.at[pl.ds(off2, RC)])

    pltpu.sync_copy(src_hbm.at[pl.ds(base, EPW)], src_v)
    pltpu.sync_copy(dst_hbm.at[pl.ds(base, EPW)], dst_v)
    ones = jnp.full((16,), 1.0, jnp.float32)
    for half in range(EPW // 16):
        ridx = lax.iota(jnp.int32, 16) + half * 16
        cidx = src_v[pl.ds(half * 16, 16)]
        plsc.store_scatter(rows_v, [ridx, cidx], ones)
    plsc.subcore_barrier()
    pltpu.sync_copy(rows_v, acc_sh.at[dst_v], add=True)
    plsc.subcore_barrier()
    off3 = pl.multiple_of(sid * RC, RC)
    pltpu.sync_copy(acc_sh.at[pl.ds(off3, RC)],
                    out_hbm.at[cid, pl.ds(off3, RC)])

    @pl.when(sid < 2)
    def _write_tail():
        off4 = pl.multiple_of(NS * RC + sid * RC, RC)
        pltpu.sync_copy(acc_sh.at[pl.ds(off4, RC)],
                        out_hbm.at[cid, pl.ds(off4, RC)])


_sc_adjacency = pl.kernel(
    _sc_build_adjacency,
    out_type=jax.ShapeDtypeStruct((NC, N, N), jnp.float32),
    mesh=plsc.VectorSubcoreMesh(core_axis_name="c", subcore_axis_name="s",
                                num_cores=NC, num_subcores=NS),
    compiler_params=pltpu.CompilerParams(use_tc_tiling_on_sc=False,
                                         needs_layout_passes=False),
    scratch_types=[
        pltpu.VMEM((EPW,), jnp.int32),
        pltpu.VMEM((EPW,), jnp.int32),
        pltpu.VMEM((EPW, N), jnp.float32),
        pltpu.VMEM_SHARED((N, N), jnp.float32),
    ],
)


def _dot(a, b):
    return jax.lax.dot_general(
        a, b, (((1,), (0,)), ((), ())), preferred_element_type=jnp.float32,
        precision=jax.lax.Precision.DEFAULT)


def _mlp_body(l_ref, w_ref, w1l_ref, b1l_ref, g_ref, bb_ref, w2l_ref,
              b2l_ref, w1w_ref, b1w_ref, w2w_ref, b2w_ref, cat_ref):
    inv = 1.0 / jnp.sqrt(1.0 + 1e-5)
    l = jnp.tanh(l_ref[...] * w1l_ref[...] + b1l_ref[...])
    l = l * (g_ref[...] * inv) + bb_ref[...]
    cat_ref[:, :H] = jnp.tanh(_dot(l, w2l_ref[...]) + b2l_ref[...])
    w = jnp.tanh(_dot(w_ref[...], w1w_ref[...]) + b1w_ref[...])
    cat_ref[:, H:] = jnp.tanh(_dot(w, w2w_ref[...]) + b2w_ref[...])


def _gnn_body(ap_ref, cat_ref, m0w_ref, m0b_ref, m1w_ref, m1b_ref,
              regr_ref, regb_ref, out_ref):
    f32 = jnp.float32
    a = ap_ref[0] + ap_ref[1]
    deg = jnp.maximum(jnp.sum(a, axis=1, keepdims=True), 1.0)
    an = a / deg
    h = _dot(an, cat_ref[...])
    l = _dot(h, m0w_ref[...]) + m0b_ref[...]
    cat1 = jnp.concatenate([l, h[:, H:]], axis=1)
    h = _dot(an, cat1)
    l = _dot(h, m1w_ref[...]) + m1b_ref[...]
    s = jnp.sum(l * regr_ref[...], axis=1, keepdims=True)
    rows = jax.lax.broadcasted_iota(jnp.int32, (BS, N), 0)
    cols = jax.lax.broadcasted_iota(jnp.int32, (BS, N), 1)
    grp = (rows == cols // NUM_LAGS).astype(f32)
    out_ref[...] = _dot(grp, s) + regb_ref[...]


def kernel(lags, weather, lags_W1, lags_b1, bn_g, bn_b, lags_W2, lags_b2,
           wea_W1, wea_b1, wea_W2, wea_b2, mgn0_W, mgn0_b, mgn1_W, mgn1_b,
           reg_W, reg_b, src, dst):
    f32 = jnp.float32
    zer = jnp.zeros((EPW, N), f32)
    ap = _sc_adjacency(src, dst, zer)
    cat = pl.pallas_call(
        _mlp_body,
        out_shape=jax.ShapeDtypeStruct((N, 2 * H), f32),
    )(lags.reshape(N, 1), weather.reshape(N, 8), lags_W1,
      lags_b1.reshape(1, H), bn_g.reshape(1, H), bn_b.reshape(1, H),
      lags_W2, lags_b2.reshape(1, H), wea_W1, wea_b1.reshape(1, H),
      wea_W2, wea_b2.reshape(1, H))
    reg_tiled = jnp.tile(reg_W.reshape(NUM_LAGS, H), (BS, 1))
    return pl.pallas_call(
        _gnn_body,
        out_shape=jax.ShapeDtypeStruct((BS, 1), f32),
    )(ap, cat, mgn0_W, mgn0_b.reshape(1, H), mgn1_W,
      mgn1_b.reshape(1, H), reg_tiled, reg_b.reshape(1, 1))


# speedup vs baseline: 1.4321x; 1.1506x over previous
import jax
import jax.numpy as jnp
from jax import lax
from jax.experimental import pallas as pl
from jax.experimental.pallas import tpu as pltpu
from jax.experimental.pallas import tpu_sc as plsc

H = 512
BS = 32
NUM_LAGS = 9
N = BS * NUM_LAGS
BE = 32
AP = 16


def _sc_build_a9(src_hbm, dst_hbm, out_hbm, src_v, dst_v, rows_v, acc_sh, sem):
    cid = lax.axis_index("c")
    sid = lax.axis_index("s")

    @pl.when(jnp.logical_and(cid == 0, sid == 0))
    def _only_tile0():
        cp1 = pltpu.async_copy(src_hbm.at[pl.ds(0, BE)], src_v, sem)
        cp2 = pltpu.async_copy(dst_hbm.at[pl.ds(0, BE)], dst_v, sem)
        z = jnp.zeros((16,), jnp.float32)
        for i in range(BE):
            rows_v[i, :] = z
        pltpu.sync_copy(rows_v.at[pl.ds(0, AP)], acc_sh)
        cp1.wait()
        cp2.wait()
        ones = jnp.full((16,), 1.0, jnp.float32)
        for half in range(BE // 16):
            ridx = lax.iota(jnp.int32, 16) + half * 16
            cidx = src_v[pl.ds(half * 16, 16)]
            plsc.store_scatter(rows_v, [ridx, cidx], ones)
        pltpu.sync_copy(rows_v, acc_sh.at[dst_v], add=True)
        pltpu.sync_copy(acc_sh, out_hbm)


_sc_adjacency = pl.kernel(
    _sc_build_a9,
    out_type=jax.ShapeDtypeStruct((AP, AP), jnp.float32),
    mesh=plsc.VectorSubcoreMesh(core_axis_name="c", subcore_axis_name="s",
                                num_cores=2, num_subcores=16),
    compiler_params=pltpu.CompilerParams(use_tc_tiling_on_sc=False,
                                         needs_layout_passes=False),
    scratch_types=[
        pltpu.VMEM((BE,), jnp.int32),
        pltpu.VMEM((BE,), jnp.int32),
        pltpu.VMEM((BE, AP), jnp.float32),
        pltpu.VMEM_SHARED((AP, AP), jnp.float32),
        pltpu.SemaphoreType.DMA,
    ],
)


def _dot(a, b):
    return jax.lax.dot_general(
        a, b, (((1,), (0,)), ((), ())), preferred_element_type=jnp.float32,
        precision=jax.lax.Precision.DEFAULT)


def _mlp_body(l_ref, w_ref, w1l_ref, b1l_ref, g_ref, bb_ref, w2l_ref,
              b2l_ref, w1w_ref, b1w_ref, w2w_ref, b2w_ref, cat_ref):
    inv = 1.0 / jnp.sqrt(1.0 + 1e-5)
    l = jnp.tanh(l_ref[...] * w1l_ref[...] + b1l_ref[...])
    l = l * (g_ref[...] * inv) + bb_ref[...]
    cat_ref[:, :H] = jnp.tanh(_dot(l, w2l_ref[...]) + b2l_ref[...])
    w = jnp.tanh(_dot(w_ref[...], w1w_ref[...]) + b1w_ref[...])
    cat_ref[:, H:] = jnp.tanh(_dot(w, w2w_ref[...]) + b2w_ref[...])


def _gnn_body(a9_ref, cat_ref, m0w_ref, m0b_ref, m1w_ref, m1b_ref,
              regr_ref, regb_ref, out_ref):
    f32 = jnp.float32
    i32 = jnp.int32
    a9 = a9_ref[...]
    deg = jnp.maximum(jnp.sum(a9, axis=1, keepdims=True), 1.0)
    a9n = a9 / deg
    c16 = lax.broadcasted_iota(i32, (N, AP), 1)
    p1 = (c16 == lax.broadcasted_iota(i32, (N, AP), 0) % NUM_LAGS)
    t1 = _dot(p1.astype(f32), a9n)
    r16 = lax.broadcasted_iota(i32, (AP, N), 0)
    p2 = (r16 == lax.broadcasted_iota(i32, (AP, N), 1) % NUM_LAGS)
    blk = (lax.broadcasted_iota(i32, (N, N), 0) // NUM_LAGS ==
           lax.broadcasted_iota(i32, (N, N), 1) // NUM_LAGS)
    an = _dot(t1, p2.astype(f32)) * blk.astype(f32)
    h = _dot(an, cat_ref[...])
    l = _dot(h, m0w_ref[...]) + m0b_ref[...]
    cat1 = jnp.concatenate([l, h[:, H:]], axis=1)
    h = _dot(an, cat1)
    l = _dot(h, m1w_ref[...]) + m1b_ref[...]
    p = jax.lax.dot_general(l, regr_ref[...], (((1,), (1,)), ((), ())),
                            preferred_element_type=f32)
    m9 = (lax.broadcasted_iota(i32, (N, NUM_LAGS), 1) ==
          lax.broadcasted_iota(i32, (N, NUM_LAGS), 0) % NUM_LAGS)
    s = jnp.sum(p * m9.astype(f32), axis=1, keepdims=True)
    rows = lax.broadcasted_iota(i32, (BS, N), 0)
    cols = lax.broadcasted_iota(i32, (BS, N), 1)
    grp = (rows == cols // NUM_LAGS).astype(f32)
    out_ref[...] = _dot(grp, s) + regb_ref[...]


def kernel(lags, weather, lags_W1, lags_b1, bn_g, bn_b, lags_W2, lags_b2,
           wea_W1, wea_b1, wea_W2, wea_b2, mgn0_W, mgn0_b, mgn1_W, mgn1_b,
           reg_W, reg_b, src, dst):
    f32 = jnp.float32
    a9 = _sc_adjacency(src, dst)
    cat = pl.pallas_call(
        _mlp_body,
        out_shape=jax.ShapeDtypeStruct((N, 2 * H), f32),
    )(lags.reshape(N, 1), weather.reshape(N, 8), lags_W1,
      lags_b1.reshape(1, H), bn_g.reshape(1, H), bn_b.reshape(1, H),
      lags_W2, lags_b2.reshape(1, H), wea_W1, wea_b1.reshape(1, H),
      wea_W2, wea_b2.reshape(1, H))
    return pl.pallas_call(
        _gnn_body,
        out_shape=jax.ShapeDtypeStruct((BS, 1), f32),
    )(a9, cat, mgn0_W, mgn0_b.reshape(1, H), mgn1_W,
      mgn1_b.reshape(1, H), reg_W.reshape(NUM_LAGS, H),
      reg_b.reshape(1, 1))


# speedup vs baseline: 1.5385x; 1.0743x over previous
import jax
import jax.numpy as jnp
from jax import lax
from jax.experimental import pallas as pl
from jax.experimental.pallas import tpu as pltpu
from jax.experimental.pallas import tpu_sc as plsc

H = 512
BS = 32
NUM_LAGS = 9
N = BS * NUM_LAGS
BE = 32
AP = 16


def _sc_build_a9(src_hbm, dst_hbm, out_hbm, src_v, dst_v, rows_v, acc_sh, sem):
    cid = lax.axis_index("c")
    sid = lax.axis_index("s")

    @pl.when(jnp.logical_and(cid == 0, sid == 0))
    def _only_tile0():
        cp1 = pltpu.async_copy(src_hbm.at[pl.ds(0, BE)], src_v, sem)
        cp2 = pltpu.async_copy(dst_hbm.at[pl.ds(0, BE)], dst_v, sem)
        z = jnp.zeros((16,), jnp.float32)
        for i in range(BE):
            rows_v[i, :] = z
        pltpu.sync_copy(rows_v.at[pl.ds(0, AP)], acc_sh)
        cp1.wait()
        cp2.wait()
        ones = jnp.full((16,), 1.0, jnp.float32)
        for half in range(BE // 16):
            ridx = lax.iota(jnp.int32, 16) + half * 16
            cidx = src_v[pl.ds(half * 16, 16)]
            plsc.store_scatter(rows_v, [ridx, cidx], ones)
        pltpu.sync_copy(rows_v, acc_sh.at[dst_v], add=True)
        pltpu.sync_copy(acc_sh, out_hbm)


_sc_adjacency = pl.kernel(
    _sc_build_a9,
    out_type=jax.ShapeDtypeStruct((AP, AP), jnp.float32),
    mesh=plsc.VectorSubcoreMesh(core_axis_name="c", subcore_axis_name="s",
                                num_cores=2, num_subcores=16),
    compiler_params=pltpu.CompilerParams(use_tc_tiling_on_sc=False,
                                         needs_layout_passes=False),
    scratch_types=[
        pltpu.VMEM((BE,), jnp.int32),
        pltpu.VMEM((BE,), jnp.int32),
        pltpu.VMEM((BE, AP), jnp.float32),
        pltpu.VMEM_SHARED((AP, AP), jnp.float32),
        pltpu.SemaphoreType.DMA,
    ],
)


def _dot(a, b):
    return jax.lax.dot_general(
        a, b, (((1,), (0,)), ((), ())), preferred_element_type=jnp.float32,
        precision=jax.lax.Precision.DEFAULT)


def _fused_body(a9_ref, l_ref, w_ref, w1l_ref, b1l_ref, g_ref, bb_ref,
                w2l_ref, b2l_ref, w1w_ref, b1w_ref, w2w_ref, b2w_ref,
                m0w_ref, m0b_ref, m1w_ref, m1b_ref, regr_ref, regb_ref,
                out_ref):
    f32 = jnp.float32
    inv = 1.0 / jnp.sqrt(1.0 + 1e-5)
    l = jnp.tanh(l_ref[...] * w1l_ref[...] + b1l_ref[...])
    l = l * (g_ref[...] * inv) + bb_ref[...]
    l = jnp.tanh(_dot(l, w2l_ref[...]) + b2l_ref[...])
    w = jnp.tanh(_dot(w_ref[...], w1w_ref[...]) + b1w_ref[...])
    w = jnp.tanh(_dot(w, w2w_ref[...]) + b2w_ref[...])
    a9 = a9_ref[...]
    deg = jnp.maximum(jnp.sum(a9, axis=1, keepdims=True), 1.0)
    a9n = (a9 / deg)[:NUM_LAGS, :NUM_LAGS]
    cat3 = jnp.reshape(jnp.concatenate([l, w], axis=1), (NUM_LAGS, BS, 2 * H))
    h3 = jax.lax.dot_general(a9n, cat3, (((1,), (0,)), ((), ())),
                             preferred_element_type=f32)
    h = jnp.reshape(h3, (N, 2 * H))
    l = _dot(h, m0w_ref[...]) + m0b_ref[...]
    cat1 = jnp.concatenate([l, h[:, H:]], axis=1)
    h3 = jax.lax.dot_general(a9n, jnp.reshape(cat1, (NUM_LAGS, BS, 2 * H)),
                             (((1,), (0,)), ((), ())),
                             preferred_element_type=f32)
    h = jnp.reshape(h3, (N, 2 * H))
    l = _dot(h, m1w_ref[...]) + m1b_ref[...]
    regt = jnp.reshape(
        jnp.broadcast_to(regr_ref[...][:, None, :], (NUM_LAGS, BS, H)),
        (N, H))
    s = jnp.sum(l * regt, axis=1, keepdims=True)
    out_ref[...] = (jnp.sum(jnp.reshape(s, (NUM_LAGS, BS, 1)), axis=0)
                    + regb_ref[...])


def kernel(lags, weather, lags_W1, lags_b1, bn_g, bn_b, lags_W2, lags_b2,
           wea_W1, wea_b1, wea_W2, wea_b2, mgn0_W, mgn0_b, mgn1_W, mgn1_b,
           reg_W, reg_b, src, dst):
    f32 = jnp.float32
    a9 = _sc_adjacency(src, dst)
    lagsT = lags.T.reshape(N, 1)
    weaT = weather.swapaxes(0, 1).reshape(N, 8)
    return pl.pallas_call(
        _fused_body,
        out_shape=jax.ShapeDtypeStruct((BS, 1), f32),
    )(a9, lagsT, weaT, lags_W1, lags_b1.reshape(1, H), bn_g.reshape(1, H),
      bn_b.reshape(1, H), lags_W2, lags_b2.reshape(1, H), wea_W1,
      wea_b1.reshape(1, H), wea_W2, wea_b2.reshape(1, H), mgn0_W,
      mgn0_b.reshape(1, H), mgn1_W, mgn1_b.reshape(1, H),
      reg_W.reshape(NUM_LAGS, H), reg_b.reshape(1, 1))


# speedup vs baseline: 1.8255x; 1.1865x over previous
import jax
import jax.numpy as jnp
from jax import lax
from jax.experimental import pallas as pl
from jax.experimental.pallas import tpu as pltpu
from jax.experimental.pallas import tpu_sc as plsc

H = 512
BS = 32
NUM_LAGS = 9
N = BS * NUM_LAGS
BE = 32
AP = 16


def _sc_build_a9(src_hbm, dst_hbm, zer_hbm, out_hbm, src_v, dst_v, rows_v,
                 acc_sh, sem):
    cid = lax.axis_index("c")
    sid = lax.axis_index("s")

    @pl.when(jnp.logical_and(cid == 0, sid == 0))
    def _only_tile0():
        cp1 = pltpu.async_copy(src_hbm.at[pl.ds(0, BE)], src_v, sem)
        cp2 = pltpu.async_copy(dst_hbm.at[pl.ds(0, BE)], dst_v, sem)
        cp3 = pltpu.async_copy(zer_hbm, rows_v, sem)
        cp3.wait()
        pltpu.sync_copy(rows_v.at[pl.ds(0, AP)], acc_sh)
        cp1.wait()
        cp2.wait()
        ones = jnp.full((16,), 1.0, jnp.float32)
        for half in range(BE // 16):
            ridx = lax.iota(jnp.int32, 16) + half * 16
            cidx = src_v[pl.ds(half * 16, 16)]
            plsc.store_scatter(rows_v, [ridx, cidx], ones)
        pltpu.sync_copy(rows_v, acc_sh.at[dst_v], add=True)
        pltpu.sync_copy(acc_sh, out_hbm.at[:, pl.ds(0, AP)])


_sc_adjacency = pl.kernel(
    _sc_build_a9,
    out_type=jax.ShapeDtypeStruct((AP, 128), jnp.float32),
    mesh=plsc.VectorSubcoreMesh(core_axis_name="c", subcore_axis_name="s",
                                num_cores=1, num_subcores=1),
    compiler_params=pltpu.CompilerParams(use_tc_tiling_on_sc=False,
                                         needs_layout_passes=False),
    scratch_types=[
        pltpu.VMEM((BE,), jnp.int32),
        pltpu.VMEM((BE,), jnp.int32),
        pltpu.VMEM((BE, AP), jnp.float32),
        pltpu.VMEM_SHARED((AP, AP), jnp.float32),
        pltpu.SemaphoreType.DMA,
    ],
)


def _dot(a, b):
    return jax.lax.dot_general(
        a, b, (((1,), (0,)), ((), ())), preferred_element_type=jnp.float32,
        precision=jax.lax.Precision.DEFAULT)


def _fused_body(a9_ref, l_ref, w_ref, w1l_ref, b1l_ref, g_ref, bb_ref,
                w2l_ref, b2l_ref, w1w_ref, b1w_ref, w2w_ref, b2w_ref,
                m0w_ref, m0b_ref, m1w_ref, m1b_ref, regr_ref, regb_ref,
                out_ref):
    f32 = jnp.float32

    def agg(a9n, x):
        x3 = jnp.reshape(x, (NUM_LAGS, BS, x.shape[1]))
        h3 = jax.lax.dot_general(a9n, x3, (((1,), (0,)), ((), ())),
                                 preferred_element_type=f32)
        return jnp.reshape(h3, (N, x.shape[1]))

    inv = 1.0 / jnp.sqrt(1.0 + 1e-5)
    l = jnp.tanh(_dot(l_ref[...], w1l_ref[...]) + b1l_ref[...])
    l = l * (g_ref[...] * inv) + bb_ref[...]
    l = jnp.tanh(_dot(l, w2l_ref[...]) + b2l_ref[...])
    w = jnp.tanh(_dot(w_ref[...], w1w_ref[...]) + b1w_ref[...])
    w = jnp.tanh(_dot(w, w2w_ref[...]) + b2w_ref[...])
    a9 = a9_ref[...][:NUM_LAGS, :NUM_LAGS]
    deg = jnp.maximum(jnp.sum(a9, axis=1, keepdims=True), 1.0)
    a9n = a9 / deg
    cat = jnp.concatenate([l, w], axis=1)
    l = agg(a9n, _dot(cat, m0w_ref[...])) + m0b_ref[...]
    w = agg(a9n, w)
    cat = jnp.concatenate([l, w], axis=1)
    l = agg(a9n, _dot(cat, m1w_ref[...])) + m1b_ref[...]
    regt = jnp.reshape(
        jnp.broadcast_to(regr_ref[...][:, None, :], (NUM_LAGS, BS, H)),
        (N, H))
    s = jnp.sum(l * regt, axis=1, keepdims=True)
    i32 = jnp.int32
    sel = (lax.broadcasted_iota(i32, (N, BS), 0) % BS ==
           lax.broadcasted_iota(i32, (N, BS), 1)).astype(f32)
    out_ref[...] = jax.lax.dot_general(
        s, sel, (((0,), (0,)), ((), ())),
        preferred_element_type=f32) + regb_ref[...]


def kernel(lags, weather, lags_W1, lags_b1, bn_g, bn_b, lags_W2, lags_b2,
           wea_W1, wea_b1, wea_W2, wea_b2, mgn0_W, mgn0_b, mgn1_W, mgn1_b,
           reg_W, reg_b, src, dst):
    f32 = jnp.float32
    zer = jnp.zeros((BE, AP), f32)
    a9 = _sc_adjacency(src, dst, zer)
    lagsT = lags.T.reshape(N, 1)
    weaT = weather.swapaxes(0, 1).reshape(N, 8)
    out = pl.pallas_call(
        _fused_body,
        out_shape=jax.ShapeDtypeStruct((1, BS), f32),
    )(a9, lagsT, weaT, lags_W1, lags_b1.reshape(1, H), bn_g.reshape(1, H),
      bn_b.reshape(1, H), lags_W2, lags_b2.reshape(1, H), wea_W1,
      wea_b1.reshape(1, H), wea_W2, wea_b2.reshape(1, H), mgn0_W,
      mgn0_b.reshape(1, H), mgn1_W, mgn1_b.reshape(1, H),
      reg_W.reshape(NUM_LAGS, H), reg_b.reshape(1, 1))
    return out.reshape(BS, 1)
